# baseline (speedup 1.0000x reference)
import functools

import jax
import jax.numpy as jnp
from jax import lax
from jax.experimental import pallas as pl
from jax.experimental.pallas import tpu as pltpu
from jax.experimental.pallas import tpu_sc as plsc


@functools.lru_cache(maxsize=None)
def _gather_call(n, b_per_w, num_cores):
    mesh = plsc.VectorSubcoreMesh(core_axis_name="c", subcore_axis_name="s")

    @functools.partial(
        pl.kernel,
        mesh=mesh,
        out_type=jax.ShapeDtypeStruct((n,), jnp.uint32),
        scratch_types=[
            pltpu.VMEM((b_per_w,), jnp.int32),
            pltpu.VMEM((b_per_w,), jnp.uint32),
            pltpu.SemaphoreType.DMA,
            pltpu.SemaphoreType.DMA,
            pltpu.SemaphoreType.DMA,
            pltpu.SemaphoreType.DMA,
            pltpu.SemaphoreType.DMA,
        ],
    )
    def k(idx_hbm, table_hbm, out_hbm, idx_v, rows_v, s0, s1, s2, s3, sw):
        wid = lax.axis_index("s") * num_cores + lax.axis_index("c")
        base = wid * b_per_w
        ck = b_per_w // 4
        pltpu.sync_copy(idx_hbm.at[pl.ds(base, b_per_w)], idx_v)
        gsems = [s0, s1, s2, s3]
        gathers = [
            pltpu.async_copy(
                table_hbm.at[idx_v.at[pl.ds(c * ck, ck)]],
                rows_v.at[pl.ds(c * ck, ck)], gsems[c])
            for c in range(4)
        ]
        writebacks = []
        for c in range(4):
            gathers[c].wait()
            writebacks.append(
                pltpu.async_copy(
                    rows_v.at[pl.ds(c * ck, ck)],
                    out_hbm.at[pl.ds(base + c * ck, ck)], sw))
        for wb in writebacks:
            wb.wait()

    return k


def kernel(ids, mapper):
    b, f = ids.shape
    n = b * f
    info = plsc.get_sparse_core_info()
    nw = info.num_cores * info.num_subcores
    b_per_w = n // nw
    idx = lax.bitcast_convert_type(ids.T.astype(jnp.uint32), jnp.int32).reshape(n)
    table = mapper.astype(jnp.uint32)
    out = _gather_call(n, b_per_w, info.num_cores)(idx, table)
    return out.astype(jnp.int64).reshape(f, b).T


# speedup vs baseline: 1.0029x; 1.0029x over previous
import functools

import jax
import jax.numpy as jnp
from jax import lax
from jax.experimental import pallas as pl
from jax.experimental.pallas import tpu as pltpu
from jax.experimental.pallas import tpu_sc as plsc


@functools.lru_cache(maxsize=None)
def _gather_call(n, b_per_w, num_cores):
    mesh = plsc.VectorSubcoreMesh(core_axis_name="c", subcore_axis_name="s")

    @functools.partial(
        pl.kernel,
        mesh=mesh,
        out_type=jax.ShapeDtypeStruct((n,), jnp.uint32),
        scratch_types=[
            pltpu.VMEM((b_per_w,), jnp.uint32),
            pltpu.VMEM((b_per_w,), jnp.uint32),
            pltpu.SemaphoreType.DMA,
        ],
    )
    def k(idx_hbm, table_hbm, out_hbm, idx_v, rows_v, sem):
        wid = lax.axis_index("s") * num_cores + lax.axis_index("c")
        base = wid * b_per_w
        pltpu.sync_copy(idx_hbm.at[pl.ds(base, b_per_w)], idx_v)
        pltpu.async_copy(table_hbm.at[idx_v], rows_v, sem).wait()
        pltpu.sync_copy(rows_v, out_hbm.at[pl.ds(base, b_per_w)])

    return k


def kernel(ids, mapper):
    b, f = ids.shape
    n = b * f
    info = plsc.get_sparse_core_info()
    nw = info.num_cores * info.num_subcores
    b_per_w = n // nw
    idx = ids.T.astype(jnp.uint32).reshape(n)
    table = mapper.astype(jnp.uint32)
    out = _gather_call(n, b_per_w, info.num_cores)(idx, table)
    return out.astype(jnp.int64).reshape(f, b).T
